# baseline (speedup 1.0000x reference)
import functools

import jax
import jax.numpy as jnp
from jax import lax
from jax.experimental import pallas as pl
from jax.experimental.pallas import tpu as pltpu
from jax.experimental.pallas import tpu_sc as plsc

NC = 2
NS = 16
NW = NC * NS
LANES = 16

N = 10000
D = 128
DEG_PAD = 10240
DEG_SLICE = DEG_PAD // NS
ACC_ROWS = 10240
RPT = ACC_ROWS // NS

CHD = 128
CHS = 96
ZRS = 80


def _sc_mesh():
    return plsc.VectorSubcoreMesh(core_axis_name="c", subcore_axis_name="s")


def _deg_body(nch, colp_hbm, out_hbm, colv, ones_v, zb, deg_sh):
    cid = lax.axis_index("c")
    sid = lax.axis_index("s")
    wid = sid * NC + cid
    pltpu.sync_copy(colp_hbm.at[wid], colv)
    def zb_body(i, _):
        zb[pl.ds(i * LANES, LANES)] = jnp.zeros((LANES,), jnp.float32)
        return 0
    lax.fori_loop(0, DEG_SLICE // LANES, zb_body, 0)
    pltpu.sync_copy(zb, deg_sh.at[pl.ds(sid * DEG_SLICE, DEG_SLICE)])
    def ones_body(i, _):
        ones_v[pl.ds(i * LANES, LANES)] = jnp.ones((LANES,), jnp.float32)
        return 0
    lax.fori_loop(0, CHD // LANES, ones_body, 0)
    plsc.subcore_barrier()
    def ch_body(c, _):
        pltpu.sync_copy(ones_v, deg_sh.at[colv.at[c]], add=True)
        return 0
    lax.fori_loop(0, nch, ch_body, 0)
    plsc.subcore_barrier()
    pltpu.sync_copy(deg_sh.at[pl.ds(sid * DEG_SLICE, DEG_SLICE)], zb)
    pltpu.sync_copy(zb, out_hbm.at[cid, pl.ds(sid * DEG_SLICE, DEG_SLICE)])


def _sc_deg(colp):
    nch = colp.shape[1]
    fn = pl.kernel(
        functools.partial(_deg_body, nch),
        out_type=jax.ShapeDtypeStruct((NC, DEG_PAD), jnp.float32),
        mesh=_sc_mesh(),
        scratch_types=[
            pltpu.VMEM((nch, CHD), jnp.int32),
            pltpu.VMEM((CHD,), jnp.float32),
            pltpu.VMEM((DEG_SLICE,), jnp.float32),
            pltpu.VMEM_SHARED((DEG_PAD,), jnp.float32),
        ],
    )
    return fn(colp)


def _quake_rsqrt(d):
    i = lax.bitcast_convert_type(d, jnp.int32)
    i = jnp.int32(0x5F3759DF) - lax.shift_right_logical(i, 1)
    y = lax.bitcast_convert_type(i, jnp.float32)
    hd = 0.5 * d
    y = y * (1.5 - hd * y * y)
    y = y * (1.5 - hd * y * y)
    return y


def _side_body(nch, xp_hbm, degp_hbm, pc_hbm, wp_hbm, out_hbm,
               pc, wv, rowsf, gidx, sidx, dinv_v, acc, dinv_sh):
    cid = lax.axis_index("c")
    sid = lax.axis_index("s")
    wid = sid * NC + cid
    pltpu.sync_copy(pc_hbm.at[wid], pc)
    pltpu.sync_copy(wp_hbm.at[wid], wv)

    nsl = DEG_PAD // NS
    pltpu.sync_copy(degp_hbm.at[0, pl.ds(sid * nsl, nsl)],
                    dinv_v.at[pl.ds(0, nsl)])
    pltpu.sync_copy(degp_hbm.at[1, pl.ds(sid * nsl, nsl)],
                    dinv_v.at[pl.ds(nsl, nsl)])
    def dinv_body(i, _):
        sl = pl.ds(i * LANES, LANES)
        d = dinv_v[sl] + dinv_v[pl.ds(nsl + i * LANES, LANES)]
        dinv_v[sl] = _quake_rsqrt(d)
        return 0
    lax.fori_loop(0, nsl // LANES, dinv_body, 0)
    pltpu.sync_copy(dinv_v.at[pl.ds(0, nsl)],
                    dinv_sh.at[pl.ds(sid * nsl, nsl)])

    def zb_body(i, _):
        for v in range(D // LANES):
            rowsf[i, pl.ds(v * LANES, LANES)] = jnp.zeros((LANES,), jnp.float32)
        return 0
    lax.fori_loop(0, ZRS, zb_body, 0)
    for k in range(RPT // ZRS):
        pltpu.sync_copy(rowsf.at[pl.ds(0, ZRS)],
                        acc.at[pl.ds(sid * RPT + k * ZRS, ZRS)])
    plsc.subcore_barrier()
    pltpu.sync_copy(dinv_sh, dinv_v)

    def ch_body(c, _):
        base = c * CHS
        for q in range(CHS // LANES):
            v = pc[pl.ds(base + q * LANES, LANES)]
            sl = pl.ds(q * LANES, LANES)
            gidx[sl] = lax.bitwise_and(v, jnp.int32(0xFFFF))
            sidx[sl] = lax.shift_right_logical(v, 16)
        pltpu.sync_copy(xp_hbm.at[gidx], rowsf)
        def scale_body(j, _):
            sl = pl.ds(j * LANES, LANES)
            dvec = plsc.load_gather(dinv_v, [gidx[sl]])
            wvec = wv[pl.ds(base + j * LANES, LANES)] * dvec
            for l in range(LANES):
                w = wvec[l]
                k = j * LANES + l
                for v in range(D // LANES):
                    vsl = pl.ds(v * LANES, LANES)
                    rowsf[k, vsl] = rowsf[k, vsl] * w
            return 0
        lax.fori_loop(0, CHS // LANES, scale_body, 0)
        pltpu.sync_copy(rowsf, acc.at[sidx], add=True)
        return 0
    lax.fori_loop(0, nch, ch_body, 0)
    plsc.subcore_barrier()

    for k in range(RPT // ZRS):
        sl = pl.ds(sid * RPT + k * ZRS, ZRS)
        pltpu.sync_copy(acc.at[sl], rowsf.at[pl.ds(0, ZRS)])
        pltpu.sync_copy(rowsf.at[pl.ds(0, ZRS)], out_hbm.at[cid, sl])


def _sc_side(xp, degp, pcp, wp):
    nch = pcp.shape[1] // CHS
    fn = pl.kernel(
        functools.partial(_side_body, nch),
        out_type=jax.ShapeDtypeStruct((NC, ACC_ROWS, D), jnp.float32),
        mesh=_sc_mesh(),
        compiler_params=pltpu.CompilerParams(needs_layout_passes=False),
        scratch_types=[
            pltpu.VMEM((pcp.shape[1],), jnp.int32),
            pltpu.VMEM((pcp.shape[1],), jnp.float32),
            pltpu.VMEM((CHS, D), jnp.float32),
            pltpu.VMEM((CHS,), jnp.int32),
            pltpu.VMEM((CHS,), jnp.int32),
            pltpu.VMEM((DEG_PAD,), jnp.float32),
            pltpu.VMEM_SHARED((ACC_ROWS, D), jnp.float32),
            pltpu.VMEM_SHARED((DEG_PAD,), jnp.float32),
        ],
    )
    return fn(xp, degp, pcp, wp)


def _dinv_block(degp_ref):
    deg = degp_ref[0] + degp_ref[1]
    return jnp.where(deg > 0, lax.rsqrt(deg), 0.0)


def _final_body(degp_ref, sp_ref, x_ref, w0_ref, w1_ref, out_ref):
    side = (sp_ref[0] + sp_ref[1]) * _dinv_block(degp_ref)
    bi = x_ref[...] * side
    dn = (((1,), (1,)), ((), ()))
    out_ref[...] = (
        lax.dot_general(side, w0_ref[...], dn, preferred_element_type=jnp.float32)
        + lax.dot_general(bi, w1_ref[...], dn, preferred_element_type=jnp.float32)
    )


def _tc_final(degp, sidep, xp, W0, W1):
    bn = 1024
    return pl.pallas_call(
        _final_body,
        grid=(DEG_PAD // bn,),
        in_specs=[
            pl.BlockSpec((NC, bn, 1), lambda i: (0, i, 0)),
            pl.BlockSpec((NC, bn, D), lambda i: (0, i, 0)),
            pl.BlockSpec((bn, D), lambda i: (i, 0)),
            pl.BlockSpec((D, D), lambda i: (0, 0)),
            pl.BlockSpec((D, D), lambda i: (0, 0)),
        ],
        out_specs=pl.BlockSpec((bn, D), lambda i: (i, 0)),
        out_shape=jax.ShapeDtypeStruct((DEG_PAD, D), jnp.float32),
    )(degp.reshape(NC, DEG_PAD, 1), sidep, xp, W0, W1)


def kernel(x, edge_index, dist_weight, W0, W1):
    row = edge_index[0].astype(jnp.int32)
    col = edge_index[1].astype(jnp.int32)
    e = row.shape[0]
    ept = e // NW

    nchd = -(-ept // CHD)
    padd = nchd * CHD - ept
    col2 = col.reshape(NW, ept)
    colp_d = jnp.pad(col2, ((0, 0), (0, padd)),
                     constant_values=N).reshape(NW, nchd, CHD)

    nchs = -(-ept // (2 * CHS)) * 2
    pads = nchs * CHS - ept
    packed = (row << 16) | col
    pcp = jnp.pad(packed.reshape(NW, ept), ((0, 0), (0, pads)))
    wp = jnp.pad(dist_weight.reshape(NW, ept), ((0, 0), (0, pads)))

    xp = jnp.pad(x, ((0, DEG_PAD - N), (0, 0)))
    degp = _sc_deg(colp_d)
    sidep = _sc_side(xp, degp, pcp, wp)
    return _tc_final(degp, sidep, xp, W0, W1)[:N]


# speedup vs baseline: 1.2025x; 1.2025x over previous
import functools

import jax
import jax.numpy as jnp
from jax import lax
from jax.experimental import pallas as pl
from jax.experimental.pallas import tpu as pltpu
from jax.experimental.pallas import tpu_sc as plsc

NC = 2
NS = 16
NW = NC * NS
LANES = 16

N = 10000
D = 128
DEG_PAD = 10240
DEG_SLICE = DEG_PAD // NS
ACC_ROWS = 10240
RPT = ACC_ROWS // NS

CHD = 128
CHS = 96
ZRS = 80


def _sc_mesh():
    return plsc.VectorSubcoreMesh(core_axis_name="c", subcore_axis_name="s")


def _deg_body(nch, colp_hbm, out_hbm, colv, ones_v, zb, deg_sh):
    cid = lax.axis_index("c")
    sid = lax.axis_index("s")
    wid = sid * NC + cid
    pltpu.sync_copy(colp_hbm.at[wid], colv)
    def zb_body(i, _):
        zb[pl.ds(i * LANES, LANES)] = jnp.zeros((LANES,), jnp.float32)
        return 0
    lax.fori_loop(0, DEG_SLICE // LANES, zb_body, 0)
    pltpu.sync_copy(zb, deg_sh.at[pl.ds(sid * DEG_SLICE, DEG_SLICE)])
    def ones_body(i, _):
        ones_v[pl.ds(i * LANES, LANES)] = jnp.ones((LANES,), jnp.float32)
        return 0
    lax.fori_loop(0, CHD // LANES, ones_body, 0)
    plsc.subcore_barrier()
    def ch_body(c, _):
        pltpu.sync_copy(ones_v, deg_sh.at[colv.at[c]], add=True)
        return 0
    lax.fori_loop(0, nch, ch_body, 0)
    plsc.subcore_barrier()
    pltpu.sync_copy(deg_sh.at[pl.ds(sid * DEG_SLICE, DEG_SLICE)], zb)
    pltpu.sync_copy(zb, out_hbm.at[cid, pl.ds(sid * DEG_SLICE, DEG_SLICE)])


def _sc_deg(colp):
    nch = colp.shape[1]
    fn = pl.kernel(
        functools.partial(_deg_body, nch),
        out_type=jax.ShapeDtypeStruct((NC, DEG_PAD), jnp.float32),
        mesh=_sc_mesh(),
        scratch_types=[
            pltpu.VMEM((nch, CHD), jnp.int32),
            pltpu.VMEM((CHD,), jnp.float32),
            pltpu.VMEM((DEG_SLICE,), jnp.float32),
            pltpu.VMEM_SHARED((DEG_PAD,), jnp.float32),
        ],
    )
    return fn(colp)


def _side_body(nch, xs_hbm, pc_hbm, wp_hbm, out_hbm,
               pc, wv, rowsf, rowsg, gidx, sidx, acc, gsem):
    cid = lax.axis_index("c")
    sid = lax.axis_index("s")
    wid = sid * NC + cid
    pltpu.sync_copy(pc_hbm.at[wid], pc)
    pltpu.sync_copy(wp_hbm.at[wid], wv)

    def zb_body(i, _):
        for v in range(D // LANES):
            rowsf[i, pl.ds(v * LANES, LANES)] = jnp.zeros((LANES,), jnp.float32)
        return 0
    lax.fori_loop(0, ZRS, zb_body, 0)
    for k in range(RPT // ZRS):
        pltpu.sync_copy(rowsf.at[pl.ds(0, ZRS)],
                        acc.at[pl.ds(sid * RPT + k * ZRS, ZRS)])
    plsc.subcore_barrier()

    rows = (rowsf, rowsg)

    def decode(cc, bb):
        base = cc * CHS
        for q in range(CHS // LANES):
            v = pc[pl.ds(base + q * LANES, LANES)]
            sl = pl.ds(q * LANES, LANES)
            gidx[bb, sl] = lax.bitwise_and(v, jnp.int32(0xFFFF))
            sidx[bb, sl] = lax.shift_right_logical(v, 16)

    def gather(bb):
        pltpu.async_copy(xs_hbm.at[gidx.at[bb]], rows[bb], gsem)

    def gather_wait(bb):
        pltpu.make_async_copy(xs_hbm.at[gidx.at[bb]], rows[bb], gsem).wait()

    def scale(cc, bb):
        def scale_body(j, _):
            wvec = wv[pl.ds(cc * CHS + j * LANES, LANES)]
            for l in range(LANES):
                w = wvec[l]
                k = j * LANES + l
                for v in range(D // LANES):
                    sl = pl.ds(v * LANES, LANES)
                    rows[bb][k, sl] = rows[bb][k, sl] * w
            return 0
        lax.fori_loop(0, CHS // LANES, scale_body, 0)

    decode(0, 0)
    gather(0)

    def pair_body(t, _):
        c0 = 2 * t
        c1 = c0 + 1
        gather_wait(0)
        decode(c1, 1)
        gather(1)
        scale(c0, 0)
        pltpu.sync_copy(rows[0], acc.at[sidx.at[0]], add=True)
        gather_wait(1)
        @pl.when(c1 + 1 < nch)
        def _():
            decode(c1 + 1, 0)
            gather(0)
        scale(c1, 1)
        pltpu.sync_copy(rows[1], acc.at[sidx.at[1]], add=True)
        return 0
    lax.fori_loop(0, nch // 2, pair_body, 0)
    plsc.subcore_barrier()

    for k in range(RPT // ZRS):
        sl = pl.ds(sid * RPT + k * ZRS, ZRS)
        pltpu.sync_copy(acc.at[sl], rowsf.at[pl.ds(0, ZRS)])
        pltpu.sync_copy(rowsf.at[pl.ds(0, ZRS)], out_hbm.at[cid, sl])


def _sc_side(xs, pcp, wp):
    nch = pcp.shape[1] // CHS
    fn = pl.kernel(
        functools.partial(_side_body, nch),
        out_type=jax.ShapeDtypeStruct((NC, ACC_ROWS, D), jnp.float32),
        mesh=_sc_mesh(),
        scratch_types=[
            pltpu.VMEM((pcp.shape[1],), jnp.int32),
            pltpu.VMEM((pcp.shape[1],), jnp.float32),
            pltpu.VMEM((CHS, D), jnp.float32),
            pltpu.VMEM((CHS, D), jnp.float32),
            pltpu.VMEM((2, CHS), jnp.int32),
            pltpu.VMEM((2, CHS), jnp.int32),
            pltpu.VMEM_SHARED((ACC_ROWS, D), jnp.float32),
            pltpu.SemaphoreType.DMA,
        ],
    )
    return fn(xs, pcp, wp)


def _dinv_block(degp_ref):
    deg = degp_ref[0] + degp_ref[1]
    return jnp.where(deg > 0, lax.rsqrt(deg), 0.0)


def _prep_body(degp_ref, x_ref, xs_ref):
    xs_ref[...] = x_ref[...] * _dinv_block(degp_ref)


def _tc_prep(degp, xp):
    bn = 1024
    return pl.pallas_call(
        _prep_body,
        grid=(DEG_PAD // bn,),
        in_specs=[
            pl.BlockSpec((NC, bn, 1), lambda i: (0, i, 0)),
            pl.BlockSpec((bn, D), lambda i: (i, 0)),
        ],
        out_specs=pl.BlockSpec((bn, D), lambda i: (i, 0)),
        out_shape=jax.ShapeDtypeStruct((DEG_PAD, D), jnp.float32),
    )(degp.reshape(NC, DEG_PAD, 1), xp)


def _final_body(degp_ref, sp_ref, x_ref, w0_ref, w1_ref, out_ref):
    side = (sp_ref[0] + sp_ref[1]) * _dinv_block(degp_ref)
    bi = x_ref[...] * side
    dn = (((1,), (1,)), ((), ()))
    out_ref[...] = (
        lax.dot_general(side, w0_ref[...], dn, preferred_element_type=jnp.float32)
        + lax.dot_general(bi, w1_ref[...], dn, preferred_element_type=jnp.float32)
    )


def _tc_final(degp, sidep, xp, W0, W1):
    bn = 1024
    return pl.pallas_call(
        _final_body,
        grid=(DEG_PAD // bn,),
        in_specs=[
            pl.BlockSpec((NC, bn, 1), lambda i: (0, i, 0)),
            pl.BlockSpec((NC, bn, D), lambda i: (0, i, 0)),
            pl.BlockSpec((bn, D), lambda i: (i, 0)),
            pl.BlockSpec((D, D), lambda i: (0, 0)),
            pl.BlockSpec((D, D), lambda i: (0, 0)),
        ],
        out_specs=pl.BlockSpec((bn, D), lambda i: (i, 0)),
        out_shape=jax.ShapeDtypeStruct((DEG_PAD, D), jnp.float32),
    )(degp.reshape(NC, DEG_PAD, 1), sidep, xp, W0, W1)


def kernel(x, edge_index, dist_weight, W0, W1):
    row = edge_index[0].astype(jnp.int32)
    col = edge_index[1].astype(jnp.int32)
    e = row.shape[0]
    ept = e // NW

    nchd = -(-ept // CHD)
    padd = nchd * CHD - ept
    col2 = col.reshape(NW, ept)
    colp_d = jnp.pad(col2, ((0, 0), (0, padd)),
                     constant_values=N).reshape(NW, nchd, CHD)

    nchs = -(-ept // (2 * CHS)) * 2
    pads = nchs * CHS - ept
    packed = (row << 16) | col
    pcp = jnp.pad(packed.reshape(NW, ept), ((0, 0), (0, pads)))
    wp = jnp.pad(dist_weight.reshape(NW, ept), ((0, 0), (0, pads)))

    xp = jnp.pad(x, ((0, DEG_PAD - N), (0, 0)))
    degp = _sc_deg(colp_d)
    xs = _tc_prep(degp, xp)
    sidep = _sc_side(xs, pcp, wp)
    return _tc_final(degp, sidep, xp, W0, W1)[:N]
